# baseline (speedup 1.0000x reference)
import functools

import jax
import jax.numpy as jnp
from jax.experimental import pallas as pl

PAD = 0
VOCAB = 100000
BATCH = 1024
BR = 16


def _onehot_body(ids_ref, out_ref):
    ids = ids_ref[:]
    cols = jax.lax.broadcasted_iota(jnp.int32, (BR, VOCAB), 1)
    hit = (cols == ids) & (ids != PAD)
    out_ref[:] = hit.astype(jnp.float32)


@jax.jit
def kernel(trg_token_ids_batch):
    grid = (BATCH // BR,)
    return pl.pallas_call(
        _onehot_body,
        grid=grid,
        in_specs=[pl.BlockSpec((BR, 1), lambda i: (i, 0))],
        out_specs=pl.BlockSpec((BR, VOCAB), lambda i: (i, 0)),
        out_shape=jax.ShapeDtypeStruct((BATCH, VOCAB), jnp.float32),
    )(trg_token_ids_batch)
